# Initial kernel scaffold; baseline (speedup 1.0000x reference)
import functools

import jax
import jax.numpy as jnp
from jax import lax
from jax.experimental import pallas as pl

K = 16
BIG = jnp.float32(3.0e38)


def _knn_body(ptsT_ref, xq_ref, edge_ref, idx_ref, *, n_total, q_blk):
    xr = ptsT_ref[0]
    xq = xq_ref[0]

    inner = lax.dot_general(
        xr, xq, (((1,), (0,)), ((), ())), preferred_element_type=jnp.float32
    )
    sqr = jnp.sum(xr * xr, axis=1, keepdims=True)
    sqq = jnp.sum(xq * xq, axis=0, keepdims=True)
    d2 = sqr + sqq - 2.0 * inner

    rows = lax.broadcasted_iota(jnp.int32, (n_total, q_blk), 0)
    qb = pl.program_id(1)
    cols = qb * q_blk + lax.broadcasted_iota(jnp.int32, (n_total, q_blk), 1)
    d2 = jnp.where(rows == cols, BIG, d2)

    d = xq.shape[0]
    edge_ref[0, 0:d, :, :] = jnp.broadcast_to(xq[:, None, :], (d, K, q_blk))

    for kk in range(K):
        m = jnp.min(d2, axis=0)
        hit = d2 == m[None, :]
        sel = jnp.where(hit, rows, n_total)
        a = jnp.min(sel, axis=0)
        onehot = rows == a[None, :]
        d2 = jnp.where(onehot, BIG, d2)
        nbr = lax.dot_general(
            xr,
            onehot.astype(jnp.float32),
            (((0,), (0,)), ((), ())),
            preferred_element_type=jnp.float32,
        )
        edge_ref[0, d : 2 * d, kk, :] = nbr - xq
        idx_ref[0, kk, :] = a


def kernel(point_cloud):
    B, d, N = point_cloud.shape
    pc_T = jnp.transpose(point_cloud, (0, 2, 1))
    q_blk = 256
    grid = (B, N // q_blk)

    edge, idx = pl.pallas_call(
        functools.partial(_knn_body, n_total=N, q_blk=q_blk),
        grid=grid,
        in_specs=[
            pl.BlockSpec((1, N, d), lambda b, q: (b, 0, 0)),
            pl.BlockSpec((1, d, q_blk), lambda b, q: (b, 0, q)),
        ],
        out_specs=[
            pl.BlockSpec((1, 2 * d, K, q_blk), lambda b, q: (b, 0, 0, q)),
            pl.BlockSpec((1, K, q_blk), lambda b, q: (b, 0, q)),
        ],
        out_shape=[
            jax.ShapeDtypeStruct((B, 2 * d, K, N), jnp.float32),
            jax.ShapeDtypeStruct((B, K, N), jnp.int32),
        ],
    )(pc_T, point_cloud)
    return edge, idx


# speedup vs baseline: 5.0144x; 5.0144x over previous
import functools

import jax
import jax.numpy as jnp
from jax import lax
from jax.experimental import pallas as pl

K = 16
BIG = 3.0e38


def _knn_body(ptsT_ref, xq_ref, edge_ref, idx_ref, *, n_total, q_blk):
    xr = ptsT_ref[0]
    xq = xq_ref[0]

    inner = lax.dot_general(
        xr.astype(jnp.bfloat16),
        xq.astype(jnp.bfloat16),
        (((1,), (0,)), ((), ())),
        preferred_element_type=jnp.float32,
    )
    sqr = jnp.sum(xr * xr, axis=1, keepdims=True)
    sqq = jnp.sum(xq * xq, axis=0, keepdims=True)
    d2 = sqr + sqq - 2.0 * inner

    rows = lax.broadcasted_iota(jnp.int32, (n_total, q_blk), 0)

    d = xq.shape[0]
    edge_ref[0, 0:d, :, :] = jnp.broadcast_to(xq[:, None, :], (d, K, q_blk))

    for kk in range(K + 1):
        m = jnp.min(d2, axis=0)
        hit = d2 == m[None, :]
        sel = jnp.where(hit, rows, n_total)
        a = jnp.min(sel, axis=0)
        onehot = rows == a[None, :]
        d2 = jnp.where(onehot, BIG, d2)
        if kk == 0:
            continue
        nbr = lax.dot_general(
            xr,
            onehot.astype(jnp.float32),
            (((0,), (0,)), ((), ())),
            precision=lax.Precision.HIGHEST,
            preferred_element_type=jnp.float32,
        )
        edge_ref[0, d : 2 * d, kk - 1, :] = nbr - xq
        idx_ref[0, kk - 1, :] = a


def kernel(point_cloud):
    B, d, N = point_cloud.shape
    pc_T = jnp.transpose(point_cloud, (0, 2, 1))
    q_blk = 256
    grid = (B, N // q_blk)

    edge, idx = pl.pallas_call(
        functools.partial(_knn_body, n_total=N, q_blk=q_blk),
        grid=grid,
        in_specs=[
            pl.BlockSpec((1, N, d), lambda b, q: (b, 0, 0)),
            pl.BlockSpec((1, d, q_blk), lambda b, q: (b, 0, q)),
        ],
        out_specs=[
            pl.BlockSpec((1, 2 * d, K, q_blk), lambda b, q: (b, 0, 0, q)),
            pl.BlockSpec((1, K, q_blk), lambda b, q: (b, 0, q)),
        ],
        out_shape=[
            jax.ShapeDtypeStruct((B, 2 * d, K, N), jnp.float32),
            jax.ShapeDtypeStruct((B, K, N), jnp.int32),
        ],
    )(pc_T, point_cloud)
    return edge, idx


# speedup vs baseline: 13.8823x; 2.7685x over previous
import functools

import jax
import jax.numpy as jnp
from jax import lax
from jax.experimental import pallas as pl
from jax.experimental.pallas import tpu as pltpu
from jax.experimental.pallas import tpu_sc as plsc

K = 16
BIG = 3.0e38


def _knn_body(ptsT_ref, xq_ref, idx_ref, *, n_total, q_blk):
    xr = ptsT_ref[0]
    xq = xq_ref[0]

    inner = lax.dot_general(
        xr.astype(jnp.bfloat16),
        xq.astype(jnp.bfloat16),
        (((1,), (0,)), ((), ())),
        preferred_element_type=jnp.float32,
    )
    sqr = jnp.sum(xr * xr, axis=1, keepdims=True)
    sqq = jnp.sum(xq * xq, axis=0, keepdims=True)
    d2 = sqr + sqq - 2.0 * inner

    rows = lax.broadcasted_iota(jnp.int32, (n_total, q_blk), 0)

    for kk in range(K + 1):
        m = jnp.min(d2, axis=0)
        hit = d2 == m[None, :]
        sel = jnp.where(hit, rows, n_total)
        a = jnp.min(sel, axis=0)
        d2 = jnp.where(rows == a[None, :], BIG, d2)
        if kk == 0:
            continue
        idx_ref[0, kk - 1, :] = a


def _topk_indices(point_cloud):
    B, d, N = point_cloud.shape
    pc_T = jnp.transpose(point_cloud, (0, 2, 1))
    q_blk = 256
    grid = (B, N // q_blk)

    return pl.pallas_call(
        functools.partial(_knn_body, n_total=N, q_blk=q_blk),
        grid=grid,
        in_specs=[
            pl.BlockSpec((1, N, d), lambda b, q: (b, 0, 0)),
            pl.BlockSpec((1, d, q_blk), lambda b, q: (b, 0, q)),
        ],
        out_specs=pl.BlockSpec((1, K, q_blk), lambda b, q: (b, 0, q)),
        out_shape=jax.ShapeDtypeStruct((B, K, N), jnp.int32),
    )(pc_T, point_cloud)


def _gather_nmc(point_cloud, idx):
    B, d, N = point_cloud.shape
    info = plsc.get_sparse_core_info()
    nw = info.num_cores * info.num_subcores
    ch = N // nw
    L = info.num_lanes
    mesh = plsc.VectorSubcoreMesh(core_axis_name="c", subcore_axis_name="s")

    @functools.partial(
        pl.kernel,
        mesh=mesh,
        out_type=jax.ShapeDtypeStruct((B, d, K, N), jnp.float32),
        compiler_params=pltpu.CompilerParams(needs_layout_passes=False),
        scratch_types=[
            pltpu.VMEM((d * N,), jnp.float32),
            pltpu.VMEM((K, ch), jnp.int32),
            pltpu.VMEM((d, K, ch), jnp.float32),
        ],
    )
    def k(pcf_hbm, idx_hbm, out_hbm, coords_v, idx_v, nmc_v):
        wid = lax.axis_index("s") * info.num_cores + lax.axis_index("c")
        n0 = wid * ch
        for b in range(B):
            pltpu.sync_copy(pcf_hbm.at[b], coords_v)
            pltpu.sync_copy(idx_hbm.at[b, :, pl.ds(n0, ch)], idx_v)

            def body(j, carry):
                for kk in range(K):
                    iv = idx_v[kk, pl.ds(j * L, L)]
                    for c in range(d):
                        nb = plsc.load_gather(coords_v, [iv + c * N])
                        cen = coords_v[pl.ds(c * N + n0 + j * L, L)]
                        nmc_v[c, kk, pl.ds(j * L, L)] = nb - cen
                return carry

            lax.fori_loop(0, ch // L, body, 0)
            pltpu.sync_copy(nmc_v, out_hbm.at[b, :, :, pl.ds(n0, ch)])

    return k(point_cloud.reshape(B, d * N), idx)


def kernel(point_cloud):
    B, d, N = point_cloud.shape
    idx = _topk_indices(point_cloud)
    nmc = _gather_nmc(point_cloud, idx)
    central = jnp.broadcast_to(point_cloud[:, :, None, :], (B, d, K, N))
    edge = jnp.concatenate([central, nmc], axis=1)
    return edge, idx


# speedup vs baseline: 17.6664x; 1.2726x over previous
import functools

import jax
import jax.numpy as jnp
from jax import lax
from jax.experimental import pallas as pl
from jax.experimental.pallas import tpu as pltpu
from jax.experimental.pallas import tpu_sc as plsc

K = 16
BIG = 3.0e38


def _knn_body(ptsT_ref, xq_ref, idx_ref, *, n_total, q_blk):
    xr = ptsT_ref[0]
    xq = xq_ref[0]

    inner = lax.dot_general(
        xr.astype(jnp.bfloat16),
        xq.astype(jnp.bfloat16),
        (((1,), (0,)), ((), ())),
        preferred_element_type=jnp.float32,
    )
    sqr = jnp.sum(xr * xr, axis=1, keepdims=True)
    sqq = jnp.sum(xq * xq, axis=0, keepdims=True)
    d2 = sqr + sqq - 2.0 * inner

    h = n_total // 2
    top = d2[:h, :]
    bot = d2[h:, :]
    iot = lax.broadcasted_iota(jnp.int32, (h, q_blk), 0)
    cmp = top <= bot
    lo = jnp.where(cmp, top, bot)
    hi = jnp.where(cmp, bot, top)
    alo = jnp.where(cmp, iot, iot + h)
    ahi = jnp.where(cmp, iot + h, iot)

    for kk in range(K + 1):
        m = jnp.min(lo, axis=0)
        a = jnp.min(jnp.where(lo == m[None, :], alo, n_total), axis=0)
        mask = alo == a[None, :]
        lo = jnp.where(mask, hi, lo)
        alo = jnp.where(mask, ahi, alo)
        hi = jnp.where(mask, BIG, hi)
        if kk == 0:
            continue
        idx_ref[0, kk - 1, :] = a


def _topk_indices(point_cloud):
    B, d, N = point_cloud.shape
    pc_T = jnp.transpose(point_cloud, (0, 2, 1))
    q_blk = 256
    grid = (B, N // q_blk)

    return pl.pallas_call(
        functools.partial(_knn_body, n_total=N, q_blk=q_blk),
        grid=grid,
        in_specs=[
            pl.BlockSpec((1, N, d), lambda b, q: (b, 0, 0)),
            pl.BlockSpec((1, d, q_blk), lambda b, q: (b, 0, q)),
        ],
        out_specs=pl.BlockSpec((1, K, q_blk), lambda b, q: (b, 0, q)),
        out_shape=jax.ShapeDtypeStruct((B, K, N), jnp.int32),
    )(pc_T, point_cloud)


def _gather_nmc(point_cloud, idx):
    B, d, N = point_cloud.shape
    info = plsc.get_sparse_core_info()
    nw = info.num_cores * info.num_subcores
    ch = N // nw
    L = info.num_lanes
    mesh = plsc.VectorSubcoreMesh(core_axis_name="c", subcore_axis_name="s")

    @functools.partial(
        pl.kernel,
        mesh=mesh,
        out_type=jax.ShapeDtypeStruct((B, d, K, N), jnp.float32),
        compiler_params=pltpu.CompilerParams(needs_layout_passes=False),
        scratch_types=[
            pltpu.VMEM((d * N,), jnp.float32),
            pltpu.VMEM((K, ch), jnp.int32),
            pltpu.VMEM((d, K, ch), jnp.float32),
        ],
    )
    def k(pcf_hbm, idx_hbm, out_hbm, coords_v, idx_v, nmc_v):
        wid = lax.axis_index("s") * info.num_cores + lax.axis_index("c")
        n0 = wid * ch
        for b in range(B):
            pltpu.sync_copy(pcf_hbm.at[b], coords_v)
            pltpu.sync_copy(idx_hbm.at[b, :, pl.ds(n0, ch)], idx_v)

            def body(j, carry):
                for kk in range(K):
                    iv = idx_v[kk, pl.ds(j * L, L)]
                    for c in range(d):
                        nb = plsc.load_gather(coords_v, [iv + c * N])
                        cen = coords_v[pl.ds(c * N + n0 + j * L, L)]
                        nmc_v[c, kk, pl.ds(j * L, L)] = nb - cen
                return carry

            lax.fori_loop(0, ch // L, body, 0)
            pltpu.sync_copy(nmc_v, out_hbm.at[b, :, :, pl.ds(n0, ch)])

    return k(point_cloud.reshape(B, d * N), idx)


def kernel(point_cloud):
    B, d, N = point_cloud.shape
    idx = _topk_indices(point_cloud)
    nmc = _gather_nmc(point_cloud, idx)
    central = jnp.broadcast_to(point_cloud[:, :, None, :], (B, d, K, N))
    edge = jnp.concatenate([central, nmc], axis=1)
    return edge, idx


# speedup vs baseline: 18.7217x; 1.0597x over previous
import functools

import jax
import jax.numpy as jnp
from jax import lax
from jax.experimental import pallas as pl
from jax.experimental.pallas import tpu as pltpu
from jax.experimental.pallas import tpu_sc as plsc

K = 16
BIG = 3.0e38


def _knn_body(ptsT_ref, xq_ref, idx_ref, *, n_total, q_blk):
    xr = ptsT_ref[0]
    xq = xq_ref[0]

    inner = lax.dot_general(
        xr.astype(jnp.bfloat16),
        xq.astype(jnp.bfloat16),
        (((1,), (0,)), ((), ())),
        preferred_element_type=jnp.float32,
    )
    sqr = jnp.sum(xr * xr, axis=1, keepdims=True)
    sqq = jnp.sum(xq * xq, axis=0, keepdims=True)
    d2 = sqr + sqq - 2.0 * inner

    h = n_total // 2
    top = d2[:h, :]
    bot = d2[h:, :]
    iot = lax.broadcasted_iota(jnp.int32, (h, q_blk), 0)
    cmp = top <= bot
    lo = jnp.where(cmp, top, bot)
    hi = jnp.where(cmp, bot, top)
    alo = jnp.where(cmp, iot, iot + h)
    ahi = jnp.where(cmp, iot + h, iot)

    for kk in range(K + 1):
        m = jnp.min(lo, axis=0)
        a = jnp.min(jnp.where(lo == m[None, :], alo, n_total), axis=0)
        mask = alo == a[None, :]
        lo = jnp.where(mask, hi, lo)
        alo = jnp.where(mask, ahi, alo)
        hi = jnp.where(mask, BIG, hi)
        if kk == 0:
            continue
        idx_ref[0, kk - 1, :] = a


def _topk_indices(point_cloud):
    B, d, N = point_cloud.shape
    pc_T = jnp.transpose(point_cloud, (0, 2, 1))
    q_blk = 512
    grid = (B, N // q_blk)

    return pl.pallas_call(
        functools.partial(_knn_body, n_total=N, q_blk=q_blk),
        grid=grid,
        in_specs=[
            pl.BlockSpec((1, N, d), lambda b, q: (b, 0, 0)),
            pl.BlockSpec((1, d, q_blk), lambda b, q: (b, 0, q)),
        ],
        out_specs=pl.BlockSpec((1, K, q_blk), lambda b, q: (b, 0, q)),
        out_shape=jax.ShapeDtypeStruct((B, K, N), jnp.int32),
    )(pc_T, point_cloud)


def _gather_nmc(point_cloud, idx):
    B, d, N = point_cloud.shape
    info = plsc.get_sparse_core_info()
    nw = info.num_cores * info.num_subcores
    ch = N // nw
    L = info.num_lanes
    mesh = plsc.VectorSubcoreMesh(core_axis_name="c", subcore_axis_name="s")

    @functools.partial(
        pl.kernel,
        mesh=mesh,
        out_type=jax.ShapeDtypeStruct((B, d, K, N), jnp.float32),
        compiler_params=pltpu.CompilerParams(needs_layout_passes=False),
        scratch_types=[
            pltpu.VMEM((d * N,), jnp.float32),
            pltpu.VMEM((K, ch), jnp.int32),
            pltpu.VMEM((d, K, ch), jnp.float32),
        ],
    )
    def k(pcf_hbm, idx_hbm, out_hbm, coords_v, idx_v, nmc_v):
        wid = lax.axis_index("s") * info.num_cores + lax.axis_index("c")
        n0 = wid * ch
        for b in range(B):
            pltpu.sync_copy(pcf_hbm.at[b], coords_v)
            pltpu.sync_copy(idx_hbm.at[b, :, pl.ds(n0, ch)], idx_v)

            def body(j, carry):
                for kk in range(K):
                    iv = idx_v[kk, pl.ds(j * L, L)]
                    for c in range(d):
                        nb = plsc.load_gather(coords_v, [iv + c * N])
                        cen = coords_v[pl.ds(c * N + n0 + j * L, L)]
                        nmc_v[c, kk, pl.ds(j * L, L)] = nb - cen
                return carry

            lax.fori_loop(0, ch // L, body, 0)
            pltpu.sync_copy(nmc_v, out_hbm.at[b, :, :, pl.ds(n0, ch)])

    return k(point_cloud.reshape(B, d * N), idx)


def kernel(point_cloud):
    B, d, N = point_cloud.shape
    idx = _topk_indices(point_cloud)
    nmc = _gather_nmc(point_cloud, idx)
    central = jnp.broadcast_to(point_cloud[:, :, None, :], (B, d, K, N))
    edge = jnp.concatenate([central, nmc], axis=1)
    return edge, idx
